# baseline (speedup 1.0000x reference)
import functools

import jax
import jax.numpy as jnp
from jax import lax
from jax.experimental import pallas as pl
from jax.experimental.pallas import tpu as pltpu
from jax.experimental.pallas import tpu_sc as plsc

_CONST = 256
_L = 16
_NC = 2
_NS = 16
_NW = _NC * _NS


def _hsum_vec(v):
    iota = lax.iota(jnp.int32, _L)
    for k in (1, 2, 4, 8):
        v = v + jnp.take(v, lax.bitwise_xor(iota, jnp.int32(k)), mode="fill")
    return v


def _rsqrt_vec(a):
    i = lax.bitcast_convert_type(a, jnp.int32)
    i = jnp.int32(0x5F3759DF) - lax.shift_right_logical(i, 1)
    y = lax.bitcast_convert_type(i, jnp.float32)
    for _ in range(4):
        y = y * (1.5 - 0.5 * a * y * y)
    return y


def kernel(s, Cr, Ci):
    B, S = s.shape
    rows_per_w = B // _NW
    IB = 128
    n_blocks = rows_per_w // IB

    col_starts = [j * _L for j in range(S // _L)]
    if S % _L:
        col_starts.append(S - _L)

    mesh = plsc.VectorSubcoreMesh(
        core_axis_name="c", subcore_axis_name="s",
        num_cores=_NC, num_subcores=_NS)

    @functools.partial(
        pl.kernel,
        out_type=(jax.ShapeDtypeStruct((S, B), jnp.float32),
                  jax.ShapeDtypeStruct((S, B), jnp.float32)),
        mesh=mesh,
        compiler_params=pltpu.CompilerParams(
            needs_layout_passes=False, use_tc_tiling_on_sc=True),
        scratch_types=[
            pltpu.VMEM((_CONST,), jnp.float32),
            pltpu.VMEM((_CONST,), jnp.float32),
            pltpu.VMEM((IB, S), jnp.int32),
            pltpu.VMEM((S, IB), jnp.float32),
            pltpu.VMEM((S, IB), jnp.float32),
        ],
    )
    def gs_kernel(s_hbm, cr_hbm, ci_hbm, re_hbm, im_hbm,
                  crt, cit, idxv, rev, imv):
        pltpu.sync_copy(cr_hbm, crt)
        pltpu.sync_copy(ci_hbm, cit)

        zero = jnp.zeros((_L,), jnp.float32)

        def acc_body(i, acc):
            ar, ai = acc
            return (ar + crt[pl.ds(i * _L, _L)], ai + cit[pl.ds(i * _L, _L)])
        ar, ai = lax.fori_loop(0, _CONST // _L, acc_body, (zero, zero))
        mr_v = _hsum_vec(ar) * (1.0 / _CONST)
        mi_v = _hsum_vec(ai) * (1.0 / _CONST)

        def en_body(i, acc):
            dr = crt[pl.ds(i * _L, _L)] - mr_v
            di = cit[pl.ds(i * _L, _L)] - mi_v
            return acc + dr * dr + di * di
        en_acc = lax.fori_loop(0, _CONST // _L, en_body, zero)
        en_v = _hsum_vec(en_acc) * (1.0 / _CONST)
        scale = jnp.where(en_v > 0.0, _rsqrt_vec(en_v), 0.0)

        def norm_body(i, _):
            sl = pl.ds(i * _L, _L)
            crt[sl] = (crt[sl] - mr_v) * scale
            cit[sl] = (cit[sl] - mi_v) * scale
            return 0
        lax.fori_loop(0, _CONST // _L, norm_body, 0)

        wid = lax.axis_index("s") * _NC + lax.axis_index("c")
        row0 = wid * rows_per_w
        iota = lax.iota(jnp.int32, _L)
        col_vecs = [j0 + iota for j0 in col_starts]

        def block_body(blk, _):
            r0 = row0 + blk * IB
            pltpu.sync_copy(s_hbm.at[pl.ds(r0, IB), :], idxv)

            def row_body(di, _):
                db = jnp.full((_L,), di, jnp.int32)
                for j0, jv in zip(col_starts, col_vecs):
                    sl = pl.ds(j0, _L)
                    idx = idxv[di, sl]
                    plsc.store_scatter(rev, [jv, db],
                                       plsc.load_gather(crt, [idx]))
                    plsc.store_scatter(imv, [jv, db],
                                       plsc.load_gather(cit, [idx]))
                return 0
            lax.fori_loop(0, IB, row_body, 0)

            pltpu.sync_copy(rev, re_hbm.at[:, pl.ds(r0, IB)])
            pltpu.sync_copy(imv, im_hbm.at[:, pl.ds(r0, IB)])
            return 0
        lax.fori_loop(0, n_blocks, block_body, 0)

    re_t, im_t = gs_kernel(s, Cr, Ci)
    return lax.complex(re_t, im_t).T


# speedup vs baseline: 1.1656x; 1.1656x over previous
import functools

import jax
import jax.numpy as jnp
from jax import lax
from jax.experimental import pallas as pl
from jax.experimental.pallas import tpu as pltpu
from jax.experimental.pallas import tpu_sc as plsc

_CONST = 256
_L = 16
_NC = 2
_NS = 16
_NW = _NC * _NS


def _hsum_vec(v):
    iota = lax.iota(jnp.int32, _L)
    for k in (1, 2, 4, 8):
        v = v + jnp.take(v, lax.bitwise_xor(iota, jnp.int32(k)), mode="fill")
    return v


def _rsqrt_vec(a):
    i = lax.bitcast_convert_type(a, jnp.int32)
    i = jnp.int32(0x5F3759DF) - lax.shift_right_logical(i, 1)
    y = lax.bitcast_convert_type(i, jnp.float32)
    for _ in range(4):
        y = y * (1.5 - 0.5 * a * y * y)
    return y


def kernel(s, Cr, Ci):
    B, S = s.shape
    rows_per_w = B // _NW
    IB = 128
    n_blocks = rows_per_w // IB

    col_starts = [j * _L for j in range(S // _L)]
    if S % _L:
        col_starts.append(S - _L)

    mesh = plsc.VectorSubcoreMesh(
        core_axis_name="c", subcore_axis_name="s",
        num_cores=_NC, num_subcores=_NS)

    @functools.partial(
        pl.kernel,
        out_type=(jax.ShapeDtypeStruct((S, B), jnp.float32),
                  jax.ShapeDtypeStruct((S, B), jnp.float32)),
        mesh=mesh,
        compiler_params=pltpu.CompilerParams(
            needs_layout_passes=False, use_tc_tiling_on_sc=True),
        scratch_types=[
            pltpu.VMEM((_CONST,), jnp.float32),
            pltpu.VMEM((_CONST,), jnp.float32),
            pltpu.VMEM((IB, S), jnp.int32),
            pltpu.VMEM((IB * (S + 1),), jnp.int32),
            pltpu.VMEM((S, IB), jnp.float32),
            pltpu.VMEM((S, IB), jnp.float32),
        ],
    )
    def gs_kernel(s_hbm, cr_hbm, ci_hbm, re_hbm, im_hbm,
                  crt, cit, idxv, idx1, rev, imv):
        pltpu.sync_copy(cr_hbm, crt)
        pltpu.sync_copy(ci_hbm, cit)

        zero = jnp.zeros((_L,), jnp.float32)

        def acc_body(i, acc):
            ar, ai = acc
            return (ar + crt[pl.ds(i * _L, _L)], ai + cit[pl.ds(i * _L, _L)])
        ar, ai = lax.fori_loop(0, _CONST // _L, acc_body, (zero, zero))
        mr_v = _hsum_vec(ar) * (1.0 / _CONST)
        mi_v = _hsum_vec(ai) * (1.0 / _CONST)

        def en_body(i, acc):
            dr = crt[pl.ds(i * _L, _L)] - mr_v
            di = cit[pl.ds(i * _L, _L)] - mi_v
            return acc + dr * dr + di * di
        en_acc = lax.fori_loop(0, _CONST // _L, en_body, zero)
        en_v = _hsum_vec(en_acc) * (1.0 / _CONST)
        scale = jnp.where(en_v > 0.0, _rsqrt_vec(en_v), 0.0)

        def norm_body(i, _):
            sl = pl.ds(i * _L, _L)
            crt[sl] = (crt[sl] - mr_v) * scale
            cit[sl] = (cit[sl] - mi_v) * scale
            return 0
        lax.fori_loop(0, _CONST // _L, norm_body, 0)

        wid = lax.axis_index("s") * _NC + lax.axis_index("c")
        row0 = wid * rows_per_w
        iota = lax.iota(jnp.int32, _L)
        SKEW = S + 1
        col_vecs = [j0 + iota for j0 in col_starts]
        gat_vecs = [(b * _L + iota) * SKEW for b in range(IB // _L)]

        def block_body(blk, _):
            r0 = row0 + blk * IB
            pltpu.sync_copy(s_hbm.at[pl.ds(r0, IB), :], idxv)

            def repack_body(di, _):
                db = jnp.full((_L,), di * SKEW, jnp.int32)
                for j0, jv in zip(col_starts, col_vecs):
                    v = idxv[di, pl.ds(j0, _L)]
                    plsc.store_scatter(idx1, [db + jv], v)
                return 0
            lax.fori_loop(0, IB, repack_body, 0)

            def col_body(j, _):
                jb = jnp.full((_L,), j, jnp.int32)
                for b, gv in enumerate(gat_vecs):
                    idx = plsc.load_gather(idx1, [gv + jb])
                    sl = pl.ds(b * _L, _L)
                    rev[j, sl] = plsc.load_gather(crt, [idx])
                    imv[j, sl] = plsc.load_gather(cit, [idx])
                return 0
            lax.fori_loop(0, S, col_body, 0)

            pltpu.sync_copy(rev, re_hbm.at[:, pl.ds(r0, IB)])
            pltpu.sync_copy(imv, im_hbm.at[:, pl.ds(r0, IB)])
            return 0
        lax.fori_loop(0, n_blocks, block_body, 0)

    re_t, im_t = gs_kernel(s, Cr, Ci)
    return lax.complex(re_t, im_t).T


# speedup vs baseline: 1.5505x; 1.3303x over previous
import functools

import jax
import jax.numpy as jnp
from jax import lax
from jax.experimental import pallas as pl
from jax.experimental.pallas import tpu as pltpu
from jax.experimental.pallas import tpu_sc as plsc

_CONST = 256
_L = 16
_NC = 2
_NS = 16
_NW = _NC * _NS


def _hsum_vec(v):
    iota = lax.iota(jnp.int32, _L)
    for k in (1, 2, 4, 8):
        v = v + jnp.take(v, lax.bitwise_xor(iota, jnp.int32(k)), mode="fill")
    return v


def _rsqrt_vec(a):
    i = lax.bitcast_convert_type(a, jnp.int32)
    i = jnp.int32(0x5F3759DF) - lax.shift_right_logical(i, 1)
    y = lax.bitcast_convert_type(i, jnp.float32)
    for _ in range(4):
        y = y * (1.5 - 0.5 * a * y * y)
    return y


def kernel(s, Cr, Ci):
    B, S = s.shape
    rows_per_w = B // _NW
    IB = 128
    n_blocks = rows_per_w // IB

    col_starts = [j * _L for j in range(S // _L)]
    if S % _L:
        col_starts.append(S - _L)

    mesh = plsc.VectorSubcoreMesh(
        core_axis_name="c", subcore_axis_name="s",
        num_cores=_NC, num_subcores=_NS)

    @functools.partial(
        pl.kernel,
        out_type=(jax.ShapeDtypeStruct((S, B), jnp.float32),
                  jax.ShapeDtypeStruct((S, B), jnp.float32)),
        mesh=mesh,
        compiler_params=pltpu.CompilerParams(
            needs_layout_passes=False, use_tc_tiling_on_sc=True),
        scratch_types=[
            pltpu.VMEM((_CONST,), jnp.float32),
            pltpu.VMEM((_CONST,), jnp.float32),
            pltpu.VMEM((IB, S), jnp.int32),
            pltpu.VMEM((IB * (S + 1),), jnp.int32),
            pltpu.VMEM((S, IB), jnp.float32),
            pltpu.VMEM((S, IB), jnp.float32),
        ],
    )
    def gs_kernel(s_hbm, cr_hbm, ci_hbm, re_hbm, im_hbm,
                  crt, cit, idxv, idx1, rev, imv):
        pltpu.sync_copy(cr_hbm, crt)
        pltpu.sync_copy(ci_hbm, cit)

        zero = jnp.zeros((_L,), jnp.float32)

        def acc_body(i, acc):
            ar, ai = acc
            return (ar + crt[pl.ds(i * _L, _L)], ai + cit[pl.ds(i * _L, _L)])
        ar, ai = lax.fori_loop(0, _CONST // _L, acc_body, (zero, zero))
        mr_v = _hsum_vec(ar) * (1.0 / _CONST)
        mi_v = _hsum_vec(ai) * (1.0 / _CONST)

        def en_body(i, acc):
            dr = crt[pl.ds(i * _L, _L)] - mr_v
            di = cit[pl.ds(i * _L, _L)] - mi_v
            return acc + dr * dr + di * di
        en_acc = lax.fori_loop(0, _CONST // _L, en_body, zero)
        en_v = _hsum_vec(en_acc) * (1.0 / _CONST)
        scale = jnp.where(en_v > 0.0, _rsqrt_vec(en_v), 0.0)

        def norm_body(i, _):
            sl = pl.ds(i * _L, _L)
            crt[sl] = (crt[sl] - mr_v) * scale
            cit[sl] = (cit[sl] - mi_v) * scale
            return 0
        lax.fori_loop(0, _CONST // _L, norm_body, 0)

        wid = lax.axis_index("s") * _NC + lax.axis_index("c")
        row0 = wid * rows_per_w
        iota = lax.iota(jnp.int32, _L)
        SKEW = S + 1
        col_vecs = [j0 + iota for j0 in col_starts]
        gat_vecs = [(b * _L + iota) * SKEW for b in range(IB // _L)]

        def block_body(blk, _):
            r0 = row0 + blk * IB
            pltpu.sync_copy(s_hbm.at[pl.ds(r0, IB), :], idxv)

            @plsc.parallel_loop(0, IB, step=1, unroll=2)
            def repack_body(di):
                db = jnp.full((_L,), di * SKEW, jnp.int32)
                for j0, jv in zip(col_starts, col_vecs):
                    v = idxv[di, pl.ds(j0, _L)]
                    plsc.store_scatter(idx1, [db + jv], v)

            @plsc.parallel_loop(0, S, step=1, unroll=2)
            def col_body(j):
                jb = jnp.full((_L,), j, jnp.int32)
                for b, gv in enumerate(gat_vecs):
                    idx = plsc.load_gather(idx1, [gv + jb])
                    sl = pl.ds(b * _L, _L)
                    rev[j, sl] = plsc.load_gather(crt, [idx])
                    imv[j, sl] = plsc.load_gather(cit, [idx])

            pltpu.sync_copy(rev, re_hbm.at[:, pl.ds(r0, IB)])
            pltpu.sync_copy(imv, im_hbm.at[:, pl.ds(r0, IB)])
            return 0
        lax.fori_loop(0, n_blocks, block_body, 0)

    re_t, im_t = gs_kernel(s, Cr, Ci)
    return lax.complex(re_t, im_t).T
